# Initial kernel scaffold; baseline (speedup 1.0000x reference)
import jax
import jax.numpy as jnp
from jax.experimental import pallas as pl


def kernel(hidden_states, gate_w, e_bias, gate_proj, up_proj, down_proj, shared_gate_w, shared_up_w, shared_down_w):
    raise NotImplementedError("write your pallas kernel here")


# speedup vs baseline: 1.7083x; 1.7083x over previous
import functools

import jax
import jax.numpy as jnp
from jax.experimental import pallas as pl
from jax.experimental.pallas import tpu as pltpu

H = 1024
I = 512
E = 8
NG = 4
T = 2048
NE = E + 1


def _routing_kernel(x_ref, gw_ref, eb_ref, comb_ref):
    x = x_ref[...]
    gw = gw_ref[...]
    logits_t = jax.lax.dot_general(gw, x, (((1,), (1,)), ((), ())),
                                   preferred_element_type=jnp.float32)
    scores = jax.nn.sigmoid(logits_t)
    sfc = scores + eb_ref[...]
    s = [sfc[e:e + 1, :] for e in range(E)]
    sc = [scores[e:e + 1, :] for e in range(E)]
    g = [s[2 * i] + s[2 * i + 1] for i in range(NG)]
    gsel = []
    for i in range(NG):
        r = jnp.zeros_like(g[0])
        for j in range(NG):
            if j == i:
                continue
            beats = (g[j] > g[i]) | ((g[j] == g[i]) & (j < i))
            r = r + beats.astype(jnp.float32)
        gsel.append(r < 2.0)
    tmp = [jnp.where(gsel[e // 2], s[e], 0.0) for e in range(E)]
    esel = []
    for e in range(E):
        r = jnp.zeros_like(g[0])
        for e2 in range(E):
            if e2 == e:
                continue
            beats = (tmp[e2] > tmp[e]) | ((tmp[e2] == tmp[e]) & (e2 < e))
            r = r + beats.astype(jnp.float32)
        esel.append(r < 2.0)
    w = [jnp.where(esel[e], sc[e], 0.0) for e in range(E)]
    wsum = w[0]
    for e in range(1, E):
        wsum = wsum + w[e]
    inv = 1.0 / (wsum + 1e-20)
    rows = [w[e] * inv for e in range(E)]
    rows.append(jnp.ones_like(w[0]))
    for _ in range(16 - NE):
        rows.append(jnp.zeros_like(w[0]))
    comb_t = jnp.concatenate(rows, axis=0)
    comb_ref[...] = comb_t.T


def _ffn_kernel(comb_ref, x_ref, wg_ref, wu_ref, wd_ref, out_ref):
    e = pl.program_id(0)
    x = x_ref[...]
    hg = jax.lax.dot_general(x, wg_ref[0], (((1,), (1,)), ((), ())),
                             preferred_element_type=jnp.float32)
    hu = jax.lax.dot_general(x, wu_ref[0], (((1,), (1,)), ((), ())),
                             preferred_element_type=jnp.float32)
    inter = (hg * jax.nn.sigmoid(hg)) * hu
    y = jax.lax.dot_general(inter.astype(jnp.bfloat16), wd_ref[0],
                            (((1,), (1,)), ((), ())),
                            preferred_element_type=jnp.float32)
    onehot = (jax.lax.broadcasted_iota(jnp.int32, (16, 1), 0) == e
              ).astype(jnp.float32)
    col = jax.lax.dot_general(comb_ref[...], onehot, (((1,), (0,)), ((), ())),
                              preferred_element_type=jnp.float32)

    @pl.when(e == 0)
    def _():
        out_ref[...] = jnp.zeros_like(out_ref)

    out_ref[...] += y * col


def kernel(hidden_states, gate_w, e_bias, gate_proj, up_proj, down_proj,
           shared_gate_w, shared_up_w, shared_down_w):
    b, ss, h = hidden_states.shape
    x = hidden_states.reshape(T, H)

    comb = pl.pallas_call(
        _routing_kernel,
        out_shape=jax.ShapeDtypeStruct((T, 16), jnp.float32),
    )(x, gate_w, e_bias.reshape(E, 1))

    bf16 = jnp.bfloat16
    wg_all = jnp.concatenate([gate_proj, shared_gate_w[None]], 0).astype(bf16)
    wu_all = jnp.concatenate([up_proj, shared_up_w[None]], 0).astype(bf16)
    wd_all = jnp.concatenate([down_proj, shared_down_w[None]], 0).astype(bf16)
    x_bf = x.astype(bf16)

    out = pl.pallas_call(
        _ffn_kernel,
        grid=(NE,),
        in_specs=[
            pl.BlockSpec((T, 16), lambda e: (0, 0)),
            pl.BlockSpec((T, H), lambda e: (0, 0)),
            pl.BlockSpec((1, I, H), lambda e: (e, 0, 0)),
            pl.BlockSpec((1, I, H), lambda e: (e, 0, 0)),
            pl.BlockSpec((1, H, I), lambda e: (e, 0, 0)),
        ],
        out_specs=pl.BlockSpec((T, H), lambda e: (0, 0)),
        out_shape=jax.ShapeDtypeStruct((T, H), jnp.float32),
    )(comb, x_bf, wg_all, wu_all, wd_all)

    return out.reshape(b, ss, h)
